# baseline (speedup 1.0000x reference)
import functools

import numpy as np
import jax
import jax.numpy as jnp
from jax import lax
from jax.experimental import pallas as pl
from jax.experimental.pallas import tpu as pltpu, tpu_sc as plsc

EMB = 16
H = 4
D_ATT = 16
B = 4096
M = 66
ROWS = B * M
G = H * B
NB = 32
PR = 66 * 128


def _proj_body(x_ref, wq_ref, wk_ref, wv_ref, wr_ref, q_ref, k_ref, v_ref, r_ref):
    x = x_ref[...]
    q_ref[...] = jnp.dot(x, wq_ref[...], preferred_element_type=jnp.float32)
    k_ref[...] = jnp.dot(x, wk_ref[...], preferred_element_type=jnp.float32)
    v_ref[...] = jnp.dot(x, wv_ref[...], preferred_element_type=jnp.float32)
    r_ref[...] = jnp.dot(x, wr_ref[...], preferred_element_type=jnp.float32)


def _attn_body(q0, q1, q2, q3, k0, k1, k2, k3, v0, v1, v2, v3, xr_ref, o_ref):
    qs = (q0, q1, q2, q3)
    ks = (k0, k1, k2, k3)
    vs = (v0, v1, v2, v3)

    def body(t, _):
        acc = jnp.zeros((M, D_ATT), dtype=jnp.float32)
        for h in range(H):
            q = qs[h][t]
            k = ks[h][t]
            v = vs[h][t]
            s = jnp.dot(q, k, preferred_element_type=jnp.float32)
            smax = jnp.max(s, axis=1, keepdims=True)
            e = jnp.exp(s - smax)
            den = jnp.sum(e, axis=1, keepdims=True)
            att = e / den
            acc = acc + jnp.dot(att, v, preferred_element_type=jnp.float32)
        o_ref[t] = jax.nn.relu(acc * (1.0 / H) + xr_ref[t])
        return 0

    jax.lax.fori_loop(0, NB, body, 0)


def _attention_layer(xf, p):
    nsteps = ROWS // PR
    q2, k2, v2, xr = pl.pallas_call(
        _proj_body,
        grid=(nsteps,),
        in_specs=[
            pl.BlockSpec((PR, EMB), lambda i: (i, 0)),
            pl.BlockSpec((EMB, H * D_ATT), lambda i: (0, 0)),
            pl.BlockSpec((EMB, H * D_ATT), lambda i: (0, 0)),
            pl.BlockSpec((EMB, H * D_ATT), lambda i: (0, 0)),
            pl.BlockSpec((EMB, D_ATT), lambda i: (0, 0)),
        ],
        out_specs=[
            pl.BlockSpec((PR, H * D_ATT), lambda i: (i, 0)),
            pl.BlockSpec((PR, H * D_ATT), lambda i: (i, 0)),
            pl.BlockSpec((PR, H * D_ATT), lambda i: (i, 0)),
            pl.BlockSpec((PR, EMB), lambda i: (i, 0)),
        ],
        out_shape=[
            jax.ShapeDtypeStruct((ROWS, H * D_ATT), jnp.float32),
            jax.ShapeDtypeStruct((ROWS, H * D_ATT), jnp.float32),
            jax.ShapeDtypeStruct((ROWS, H * D_ATT), jnp.float32),
            jax.ShapeDtypeStruct((ROWS, EMB), jnp.float32),
        ],
    )(xf, p['wq'].T, p['wk'].T, p['wv'].T, p['wr'].T)

    q3 = q2.reshape(G, M, D_ATT)
    k3 = k2.reshape(G, D_ATT, M)
    v3 = v2.reshape(G, M, D_ATT)
    xr3 = xr.reshape(B, M, EMB)

    nblk = B // NB
    qspec = [pl.BlockSpec((NB, M, D_ATT), (lambda i, h=h: (h * nblk + i, 0, 0)))
             for h in range(H)]
    kspec = [pl.BlockSpec((NB, D_ATT, M), (lambda i, h=h: (h * nblk + i, 0, 0)))
             for h in range(H)]
    vspec = [pl.BlockSpec((NB, M, D_ATT), (lambda i, h=h: (h * nblk + i, 0, 0)))
             for h in range(H)]
    out3 = pl.pallas_call(
        _attn_body,
        grid=(nblk,),
        in_specs=qspec + kspec + vspec + [
            pl.BlockSpec((NB, M, EMB), lambda i: (i, 0, 0)),
        ],
        out_specs=pl.BlockSpec((NB, M, EMB), lambda i: (i, 0, 0)),
        out_shape=jax.ShapeDtypeStruct((B, M, EMB), jnp.float32),
    )(q3, q3, q3, q3, k3, k3, k3, k3, v3, v3, v3, v3, xr3)
    return out3.reshape(ROWS, EMB)


NW = 32
BPW = B // NW
NG = 118

_sc_mesh = plsc.VectorSubcoreMesh(core_axis_name="c", subcore_axis_name="s")


@functools.partial(
    pl.kernel,
    out_type=jax.ShapeDtypeStruct((B, 64, EMB), jnp.float32),
    mesh=_sc_mesh,
    scratch_types=[
        pltpu.VMEM((NG, BPW), jnp.int32),
        pltpu.VMEM((2, BPW, EMB), jnp.float32),
        pltpu.VMEM((2, BPW, EMB), jnp.float32),
        pltpu.VMEM((BPW, EMB), jnp.float32),
        pltpu.SemaphoreType.DMA,
        pltpu.SemaphoreType.DMA,
    ],
    compiler_params=pltpu.CompilerParams(use_tc_tiling_on_sc=False),
)
def _sc_fields(tbl, idx, out, idxall, bufa, bufb, acc, sema, semb):
    c = lax.axis_index("c")
    s = lax.axis_index("s")
    wid = s * 2 + c
    base = wid * BPW
    pltpu.sync_copy(idx.at[:, pl.ds(base, BPW)], idxall)

    pltpu.async_copy(tbl.at[idxall.at[0]], bufa.at[0], sema)
    pltpu.async_copy(tbl.at[idxall.at[45]], bufb.at[0], semb)

    def pair_body(p, _):
        cur = lax.rem(p, 2)
        pltpu.make_async_copy(tbl.at[idxall.at[p]], bufa.at[cur], sema).wait()
        pltpu.make_async_copy(tbl.at[idxall.at[45 + p]], bufb.at[cur], semb).wait()

        @pl.when(p < 44)
        def _():
            pltpu.async_copy(tbl.at[idxall.at[p + 1]], bufa.at[1 - cur], sema)
            pltpu.async_copy(tbl.at[idxall.at[46 + p]], bufb.at[1 - cur], semb)

        def mul(i, _):
            acc[i] = bufa[cur, i] * bufb[cur, i]
            return 0
        lax.fori_loop(0, BPW, mul, 0)
        pltpu.sync_copy(acc, out.at[pl.ds(base, BPW), 9 + p])
        return 0
    lax.fori_loop(0, 45, pair_body, 0)

    pltpu.async_copy(tbl.at[idxall.at[90]], bufa.at[0], sema)

    def single_body(si, _):
        cur = lax.rem(si, 2)
        pltpu.make_async_copy(tbl.at[idxall.at[90 + si]], bufa.at[cur], sema).wait()

        @pl.when(si < 17)
        def _():
            pltpu.async_copy(tbl.at[idxall.at[91 + si]], bufa.at[1 - cur], sema)
        f = jnp.where(si < 9, si, 45 + si)
        pltpu.sync_copy(bufa.at[cur], out.at[pl.ds(base, BPW), f])
        return 0
    lax.fori_loop(0, 18, single_body, 0)

    pltpu.async_copy(tbl.at[idxall.at[108]], bufb.at[0], semb)

    def title_body(ti, _):
        cur = lax.rem(ti, 2)
        pltpu.make_async_copy(tbl.at[idxall.at[108 + ti]], bufb.at[cur], semb).wait()

        @pl.when(ti < 9)
        def _():
            pltpu.async_copy(tbl.at[idxall.at[109 + ti]], bufb.at[1 - cur], semb)

        def addrow(i, _):
            @pl.when(ti == 0)
            def _():
                acc[i] = bufb[cur, i]

            @pl.when(ti > 0)
            def _():
                acc[i] = acc[i] + bufb[cur, i]
            return 0
        lax.fori_loop(0, BPW, addrow, 0)
        return 0
    lax.fori_loop(0, 10, title_body, 0)

    def scalerow(i, _):
        acc[i] = acc[i] * 0.1
        return 0
    lax.fori_loop(0, BPW, scalerow, 0)
    pltpu.sync_copy(acc, out.at[pl.ds(base, BPW), 63])


def _gather_fields(x, emb1, pair_tables, emb3, title_table):
    xi_t = x[:, :28].astype(jnp.int32).T
    tables = ([pt[0] for pt in pair_tables] + [pt[1] for pt in pair_tables]
              + list(emb1) + list(emb3) + [title_table])
    pairs = [(i, j) for i in range(9) for j in range(i, 9)]
    cols = ([i for i, _ in pairs] + [j for _, j in pairs]
            + list(range(9)) + list(range(9, 18)) + list(range(18, 28)))
    sizes = [int(t.shape[0]) for t in tables]
    offs = np.concatenate([[0], np.cumsum(sizes[:-1])]).astype(np.int32)
    row_tbl = list(range(108)) + [108] * 10
    row_off = offs[np.array(row_tbl)]
    big = jnp.concatenate(tables, axis=0)
    idx = xi_t[np.array(cols)] + jnp.asarray(row_off)[:, None]
    return _sc_fields(big, idx)


def _va_body(xd_ref, vw_ref, vb_ref, aw_ref, ab_ref, o_ref):
    xd = xd_ref[...]
    va = jnp.dot(xd[:, :128], vw_ref[...], preferred_element_type=jnp.float32)
    aa = jnp.dot(xd[:, 128:], aw_ref[...], preferred_element_type=jnp.float32)
    o_ref[...] = jnp.concatenate([va + vb_ref[...], aa + ab_ref[...]], axis=1)


def _final_body(x_ref, w_ref, b_ref, o_ref):
    o_ref[...] = jnp.sum(x_ref[...] * w_ref[...], axis=1, keepdims=True) + b_ref[0, 0]


def kernel(x, emb1, pair_tables, emb3, title_table, video_W, video_b,
           audio_W, audio_b, att1, att2, lin_W, lin_b):
    fields = _gather_fields(x, emb1, pair_tables, emb3, title_table)

    va = pl.pallas_call(
        _va_body,
        out_shape=jax.ShapeDtypeStruct((B, 2 * EMB), jnp.float32),
    )(x[:, 28:284], video_W.T, video_b[None, :], audio_W.T, audio_b[None, :])

    xf = jnp.concatenate([fields, va.reshape(B, 2, EMB)], axis=1).reshape(ROWS, EMB)

    xf = _attention_layer(xf, att1)
    xf = _attention_layer(xf, att2)
    xf = _attention_layer(xf, att2)

    return pl.pallas_call(
        _final_body,
        out_shape=jax.ShapeDtypeStruct((B, 1), jnp.float32),
    )(xf.reshape(B, M * EMB), lin_W, lin_b[None, :])


# speedup vs baseline: 1.0472x; 1.0472x over previous
import functools

import numpy as np
import jax
import jax.numpy as jnp
from jax import lax
from jax.experimental import pallas as pl
from jax.experimental.pallas import tpu as pltpu, tpu_sc as plsc

EMB = 16
H = 4
D_ATT = 16
B = 4096
M = 66
ROWS = B * M
G = H * B
NB = 32
PR = 66 * 128


def _proj_body(x_ref, w_ref, q_ref, k_ref, v_ref, r_ref):
    y = jnp.dot(x_ref[...], w_ref[...], preferred_element_type=jnp.float32)
    q_ref[...] = y[:, 0:64]
    k_ref[...] = y[:, 64:128]
    v_ref[...] = y[:, 128:192]
    r_ref[...] = y[:, 192:208]


def _attn_body(q0, q1, q2, q3, k0, k1, k2, k3, v0, v1, v2, v3, xr_ref, o_ref):
    qs = (q0, q1, q2, q3)
    ks = (k0, k1, k2, k3)
    vs = (v0, v1, v2, v3)
    zk = jnp.zeros((D_ATT, M), jnp.float32)
    zv = jnp.zeros((M, D_ATT), jnp.bfloat16)

    def body(t, _):
        a = 2 * t
        b = 2 * t + 1
        acc = jnp.zeros((M, 2 * D_ATT), dtype=jnp.float32)
        for h in range(H):
            qcat = jnp.concatenate([qs[h][a], qs[h][b]], axis=1)
            kblk = jnp.concatenate([
                jnp.concatenate([ks[h][a], zk], axis=1),
                jnp.concatenate([zk, ks[h][b]], axis=1)], axis=0)
            s = jnp.dot(qcat, kblk, preferred_element_type=jnp.float32)
            sa = s[:, :M]
            sb = s[:, M:]
            ea = jnp.exp(sa - jnp.max(sa, axis=1, keepdims=True))
            eb = jnp.exp(sb - jnp.max(sb, axis=1, keepdims=True))
            aa = ea / jnp.sum(ea, axis=1, keepdims=True)
            ab = eb / jnp.sum(eb, axis=1, keepdims=True)
            att = jnp.concatenate([aa, ab], axis=1).astype(jnp.bfloat16)
            vblk = jnp.concatenate([
                jnp.concatenate([vs[h][a].astype(jnp.bfloat16), zv], axis=1),
                jnp.concatenate([zv, vs[h][b].astype(jnp.bfloat16)], axis=1)],
                axis=0)
            acc = acc + jnp.dot(att, vblk, preferred_element_type=jnp.float32)
        o_ref[a] = jax.nn.relu(acc[:, :D_ATT] * (1.0 / H) + xr_ref[a])
        o_ref[b] = jax.nn.relu(acc[:, D_ATT:] * (1.0 / H) + xr_ref[b])
        return 0

    jax.lax.fori_loop(0, NB // 2, body, 0)


def _attention_layer(xf, p):
    nsteps = ROWS // PR
    wall = jnp.concatenate([p['wq'].T, p['wk'].T, p['wv'].T, p['wr'].T], axis=1)
    q2, k2, v2, xr = pl.pallas_call(
        _proj_body,
        grid=(nsteps,),
        in_specs=[
            pl.BlockSpec((PR, EMB), lambda i: (i, 0)),
            pl.BlockSpec((EMB, 208), lambda i: (0, 0)),
        ],
        out_specs=[
            pl.BlockSpec((PR, H * D_ATT), lambda i: (i, 0)),
            pl.BlockSpec((PR, H * D_ATT), lambda i: (i, 0)),
            pl.BlockSpec((PR, H * D_ATT), lambda i: (i, 0)),
            pl.BlockSpec((PR, EMB), lambda i: (i, 0)),
        ],
        out_shape=[
            jax.ShapeDtypeStruct((ROWS, H * D_ATT), jnp.float32),
            jax.ShapeDtypeStruct((ROWS, H * D_ATT), jnp.float32),
            jax.ShapeDtypeStruct((ROWS, H * D_ATT), jnp.float32),
            jax.ShapeDtypeStruct((ROWS, EMB), jnp.float32),
        ],
    )(xf, wall)

    q3 = q2.reshape(G, M, D_ATT)
    k3 = k2.reshape(G, D_ATT, M)
    v3 = v2.reshape(G, M, D_ATT)
    xr3 = xr.reshape(B, M, EMB)

    nblk = B // NB
    qspec = [pl.BlockSpec((NB, M, D_ATT), (lambda i, h=h: (h * nblk + i, 0, 0)))
             for h in range(H)]
    kspec = [pl.BlockSpec((NB, D_ATT, M), (lambda i, h=h: (h * nblk + i, 0, 0)))
             for h in range(H)]
    vspec = [pl.BlockSpec((NB, M, D_ATT), (lambda i, h=h: (h * nblk + i, 0, 0)))
             for h in range(H)]
    out3 = pl.pallas_call(
        _attn_body,
        grid=(nblk,),
        in_specs=qspec + kspec + vspec + [
            pl.BlockSpec((NB, M, EMB), lambda i: (i, 0, 0)),
        ],
        out_specs=pl.BlockSpec((NB, M, EMB), lambda i: (i, 0, 0)),
        out_shape=jax.ShapeDtypeStruct((B, M, EMB), jnp.float32),
    )(q3, q3, q3, q3, k3, k3, k3, k3, v3, v3, v3, v3, xr3)
    return out3.reshape(ROWS, EMB)


NW = 32
BPW = B // NW
NG = 118

_sc_mesh = plsc.VectorSubcoreMesh(core_axis_name="c", subcore_axis_name="s")


@functools.partial(
    pl.kernel,
    out_type=jax.ShapeDtypeStruct((B, 64, EMB), jnp.float32),
    mesh=_sc_mesh,
    scratch_types=[
        pltpu.VMEM((NG, BPW), jnp.int32),
        pltpu.VMEM((2, BPW, EMB), jnp.float32),
        pltpu.VMEM((2, BPW, EMB), jnp.float32),
        pltpu.VMEM((BPW, EMB), jnp.float32),
        pltpu.SemaphoreType.DMA,
        pltpu.SemaphoreType.DMA,
    ],
    compiler_params=pltpu.CompilerParams(use_tc_tiling_on_sc=False),
)
def _sc_fields(tbl, idx, out, idxall, bufa, bufb, acc, sema, semb):
    c = lax.axis_index("c")
    s = lax.axis_index("s")
    wid = s * 2 + c
    base = wid * BPW
    pltpu.sync_copy(idx.at[:, pl.ds(base, BPW)], idxall)

    pltpu.async_copy(tbl.at[idxall.at[0]], bufa.at[0], sema)
    pltpu.async_copy(tbl.at[idxall.at[45]], bufb.at[0], semb)

    def pair_body(p, _):
        cur = lax.rem(p, 2)
        pltpu.make_async_copy(tbl.at[idxall.at[p]], bufa.at[cur], sema).wait()
        pltpu.make_async_copy(tbl.at[idxall.at[45 + p]], bufb.at[cur], semb).wait()

        @pl.when(p < 44)
        def _():
            pltpu.async_copy(tbl.at[idxall.at[p + 1]], bufa.at[1 - cur], sema)
            pltpu.async_copy(tbl.at[idxall.at[46 + p]], bufb.at[1 - cur], semb)

        def mul(i, _):
            acc[i] = bufa[cur, i] * bufb[cur, i]
            return 0
        lax.fori_loop(0, BPW, mul, 0)
        pltpu.sync_copy(acc, out.at[pl.ds(base, BPW), 9 + p])
        return 0
    lax.fori_loop(0, 45, pair_body, 0)

    pltpu.async_copy(tbl.at[idxall.at[90]], bufa.at[0], sema)

    def single_body(si, _):
        cur = lax.rem(si, 2)
        pltpu.make_async_copy(tbl.at[idxall.at[90 + si]], bufa.at[cur], sema).wait()

        @pl.when(si < 17)
        def _():
            pltpu.async_copy(tbl.at[idxall.at[91 + si]], bufa.at[1 - cur], sema)
        f = jnp.where(si < 9, si, 45 + si)
        pltpu.sync_copy(bufa.at[cur], out.at[pl.ds(base, BPW), f])
        return 0
    lax.fori_loop(0, 18, single_body, 0)

    pltpu.async_copy(tbl.at[idxall.at[108]], bufb.at[0], semb)

    def title_body(ti, _):
        cur = lax.rem(ti, 2)
        pltpu.make_async_copy(tbl.at[idxall.at[108 + ti]], bufb.at[cur], semb).wait()

        @pl.when(ti < 9)
        def _():
            pltpu.async_copy(tbl.at[idxall.at[109 + ti]], bufb.at[1 - cur], semb)

        def addrow(i, _):
            @pl.when(ti == 0)
            def _():
                acc[i] = bufb[cur, i]

            @pl.when(ti > 0)
            def _():
                acc[i] = acc[i] + bufb[cur, i]
            return 0
        lax.fori_loop(0, BPW, addrow, 0)
        return 0
    lax.fori_loop(0, 10, title_body, 0)

    def scalerow(i, _):
        acc[i] = acc[i] * 0.1
        return 0
    lax.fori_loop(0, BPW, scalerow, 0)
    pltpu.sync_copy(acc, out.at[pl.ds(base, BPW), 63])


def _gather_fields(x, emb1, pair_tables, emb3, title_table):
    xi_t = x[:, :28].astype(jnp.int32).T
    tables = ([pt[0] for pt in pair_tables] + [pt[1] for pt in pair_tables]
              + list(emb1) + list(emb3) + [title_table])
    pairs = [(i, j) for i in range(9) for j in range(i, 9)]
    cols = ([i for i, _ in pairs] + [j for _, j in pairs]
            + list(range(9)) + list(range(9, 18)) + list(range(18, 28)))
    sizes = [int(t.shape[0]) for t in tables]
    offs = np.concatenate([[0], np.cumsum(sizes[:-1])]).astype(np.int32)
    row_tbl = list(range(108)) + [108] * 10
    row_off = offs[np.array(row_tbl)]
    big = jnp.concatenate(tables, axis=0)
    idx = xi_t[np.array(cols)] + jnp.asarray(row_off)[:, None]
    return _sc_fields(big, idx)


def _va_body(xd_ref, vw_ref, vb_ref, aw_ref, ab_ref, o_ref):
    xd = xd_ref[...]
    va = jnp.dot(xd[:, :128], vw_ref[...], preferred_element_type=jnp.float32)
    aa = jnp.dot(xd[:, 128:], aw_ref[...], preferred_element_type=jnp.float32)
    o_ref[...] = jnp.concatenate([va + vb_ref[...], aa + ab_ref[...]], axis=1)


def _final_body(x_ref, w_ref, b_ref, o_ref):
    o_ref[...] = jnp.sum(x_ref[...] * w_ref[...], axis=1, keepdims=True) + b_ref[0, 0]


def kernel(x, emb1, pair_tables, emb3, title_table, video_W, video_b,
           audio_W, audio_b, att1, att2, lin_W, lin_b):
    fields = _gather_fields(x, emb1, pair_tables, emb3, title_table)

    va = pl.pallas_call(
        _va_body,
        out_shape=jax.ShapeDtypeStruct((B, 2 * EMB), jnp.float32),
    )(x[:, 28:284], video_W.T, video_b[None, :], audio_W.T, audio_b[None, :])

    xf = jnp.concatenate([fields, va.reshape(B, 2, EMB)], axis=1).reshape(ROWS, EMB)

    xf = _attention_layer(xf, att1)
    xf = _attention_layer(xf, att2)
    xf = _attention_layer(xf, att2)

    return pl.pallas_call(
        _final_body,
        out_shape=jax.ShapeDtypeStruct((B, 1), jnp.float32),
    )(xf.reshape(B, M * EMB), lin_W, lin_b[None, :])


# speedup vs baseline: 1.0934x; 1.0441x over previous
import functools

import numpy as np
import jax
import jax.numpy as jnp
from jax import lax
from jax.experimental import pallas as pl
from jax.experimental.pallas import tpu as pltpu, tpu_sc as plsc

EMB = 16
H = 4
D_ATT = 16
B = 4096
M = 66
ROWS = B * M
G = H * B
NB = 32
PR = 66 * 128


def _proj_body(x_ref, w_ref, q_ref, k_ref, v_ref, r_ref):
    y = jnp.dot(x_ref[...], w_ref[...], preferred_element_type=jnp.float32)
    q_ref[...] = y[:, 0:64]
    k_ref[...] = y[:, 64:128]
    v_ref[...] = y[:, 128:192]
    r_ref[...] = y[:, 192:208]


def _attn_body(q0, q1, q2, q3, k0, k1, k2, k3, v0, v1, v2, v3, xr_ref, o_ref):
    qs = (q0, q1, q2, q3)
    ks = (k0, k1, k2, k3)
    vs = (v0, v1, v2, v3)
    zk = jnp.zeros((D_ATT, M), jnp.float32)
    zv = jnp.zeros((M, D_ATT), jnp.bfloat16)

    UNROLL = 4

    def body(t, _):
        for u in range(UNROLL):
            a = 2 * (UNROLL * t + u)
            b = a + 1
            acc = jnp.zeros((M, 2 * D_ATT), dtype=jnp.float32)
            for h in range(H):
                qcat = jnp.concatenate([qs[h][a], qs[h][b]], axis=1)
                kblk = jnp.concatenate([
                    jnp.concatenate([ks[h][a], zk], axis=1),
                    jnp.concatenate([zk, ks[h][b]], axis=1)], axis=0)
                s = jnp.dot(qcat, kblk, preferred_element_type=jnp.float32)
                sa = s[:, :M]
                sb = s[:, M:]
                ea = jnp.exp(sa - jnp.max(sa, axis=1, keepdims=True))
                eb = jnp.exp(sb - jnp.max(sb, axis=1, keepdims=True))
                aa = ea / jnp.sum(ea, axis=1, keepdims=True)
                ab = eb / jnp.sum(eb, axis=1, keepdims=True)
                att = jnp.concatenate([aa, ab], axis=1).astype(jnp.bfloat16)
                vblk = jnp.concatenate([
                    jnp.concatenate([vs[h][a].astype(jnp.bfloat16), zv], axis=1),
                    jnp.concatenate([zv, vs[h][b].astype(jnp.bfloat16)], axis=1)],
                    axis=0)
                acc = acc + jnp.dot(att, vblk, preferred_element_type=jnp.float32)
            o_ref[a] = jax.nn.relu(acc[:, :D_ATT] * (1.0 / H) + xr_ref[a])
            o_ref[b] = jax.nn.relu(acc[:, D_ATT:] * (1.0 / H) + xr_ref[b])
        return 0

    jax.lax.fori_loop(0, NB // (2 * UNROLL), body, 0)


def _attention_layer(xf, p):
    nsteps = ROWS // PR
    wall = jnp.concatenate([p['wq'].T, p['wk'].T, p['wv'].T, p['wr'].T], axis=1)
    q2, k2, v2, xr = pl.pallas_call(
        _proj_body,
        grid=(nsteps,),
        in_specs=[
            pl.BlockSpec((PR, EMB), lambda i: (i, 0)),
            pl.BlockSpec((EMB, 208), lambda i: (0, 0)),
        ],
        out_specs=[
            pl.BlockSpec((PR, H * D_ATT), lambda i: (i, 0)),
            pl.BlockSpec((PR, H * D_ATT), lambda i: (i, 0)),
            pl.BlockSpec((PR, H * D_ATT), lambda i: (i, 0)),
            pl.BlockSpec((PR, EMB), lambda i: (i, 0)),
        ],
        out_shape=[
            jax.ShapeDtypeStruct((ROWS, H * D_ATT), jnp.float32),
            jax.ShapeDtypeStruct((ROWS, H * D_ATT), jnp.float32),
            jax.ShapeDtypeStruct((ROWS, H * D_ATT), jnp.float32),
            jax.ShapeDtypeStruct((ROWS, EMB), jnp.float32),
        ],
    )(xf, wall)

    q3 = q2.reshape(G, M, D_ATT)
    k3 = k2.reshape(G, D_ATT, M)
    v3 = v2.reshape(G, M, D_ATT)
    xr3 = xr.reshape(B, M, EMB)

    nblk = B // NB
    qspec = [pl.BlockSpec((NB, M, D_ATT), (lambda i, h=h: (h * nblk + i, 0, 0)))
             for h in range(H)]
    kspec = [pl.BlockSpec((NB, D_ATT, M), (lambda i, h=h: (h * nblk + i, 0, 0)))
             for h in range(H)]
    vspec = [pl.BlockSpec((NB, M, D_ATT), (lambda i, h=h: (h * nblk + i, 0, 0)))
             for h in range(H)]
    out3 = pl.pallas_call(
        _attn_body,
        grid=(nblk,),
        in_specs=qspec + kspec + vspec + [
            pl.BlockSpec((NB, M, EMB), lambda i: (i, 0, 0)),
        ],
        out_specs=pl.BlockSpec((NB, M, EMB), lambda i: (i, 0, 0)),
        out_shape=jax.ShapeDtypeStruct((B, M, EMB), jnp.float32),
    )(q3, q3, q3, q3, k3, k3, k3, k3, v3, v3, v3, v3, xr3)
    return out3.reshape(ROWS, EMB)


NW = 32
BPW = B // NW
NG = 118

_sc_mesh = plsc.VectorSubcoreMesh(core_axis_name="c", subcore_axis_name="s")


@functools.partial(
    pl.kernel,
    out_type=jax.ShapeDtypeStruct((B, 64, EMB), jnp.float32),
    mesh=_sc_mesh,
    scratch_types=[
        pltpu.VMEM((NG, BPW), jnp.int32),
        pltpu.VMEM((2, BPW, EMB), jnp.float32),
        pltpu.VMEM((2, BPW, EMB), jnp.float32),
        pltpu.VMEM((BPW, EMB), jnp.float32),
        pltpu.SemaphoreType.DMA,
        pltpu.SemaphoreType.DMA,
    ],
    compiler_params=pltpu.CompilerParams(use_tc_tiling_on_sc=False),
)
def _sc_fields(tbl, idx, out, idxall, bufa, bufb, acc, sema, semb):
    c = lax.axis_index("c")
    s = lax.axis_index("s")
    wid = s * 2 + c
    base = wid * BPW
    pltpu.sync_copy(idx.at[:, pl.ds(base, BPW)], idxall)

    pltpu.async_copy(tbl.at[idxall.at[0]], bufa.at[0], sema)
    pltpu.async_copy(tbl.at[idxall.at[45]], bufb.at[0], semb)

    def pair_body(p, _):
        cur = lax.rem(p, 2)
        pltpu.make_async_copy(tbl.at[idxall.at[p]], bufa.at[cur], sema).wait()
        pltpu.make_async_copy(tbl.at[idxall.at[45 + p]], bufb.at[cur], semb).wait()

        @pl.when(p < 44)
        def _():
            pltpu.async_copy(tbl.at[idxall.at[p + 1]], bufa.at[1 - cur], sema)
            pltpu.async_copy(tbl.at[idxall.at[46 + p]], bufb.at[1 - cur], semb)

        def mul(i, _):
            acc[i] = bufa[cur, i] * bufb[cur, i]
            return 0
        lax.fori_loop(0, BPW, mul, 0)
        pltpu.sync_copy(acc, out.at[pl.ds(base, BPW), 9 + p])
        return 0
    lax.fori_loop(0, 45, pair_body, 0)

    pltpu.async_copy(tbl.at[idxall.at[90]], bufa.at[0], sema)

    def single_body(si, _):
        cur = lax.rem(si, 2)
        pltpu.make_async_copy(tbl.at[idxall.at[90 + si]], bufa.at[cur], sema).wait()

        @pl.when(si < 17)
        def _():
            pltpu.async_copy(tbl.at[idxall.at[91 + si]], bufa.at[1 - cur], sema)
        f = jnp.where(si < 9, si, 45 + si)
        pltpu.sync_copy(bufa.at[cur], out.at[pl.ds(base, BPW), f])
        return 0
    lax.fori_loop(0, 18, single_body, 0)

    pltpu.async_copy(tbl.at[idxall.at[108]], bufb.at[0], semb)

    def title_body(ti, _):
        cur = lax.rem(ti, 2)
        pltpu.make_async_copy(tbl.at[idxall.at[108 + ti]], bufb.at[cur], semb).wait()

        @pl.when(ti < 9)
        def _():
            pltpu.async_copy(tbl.at[idxall.at[109 + ti]], bufb.at[1 - cur], semb)

        def addrow(i, _):
            @pl.when(ti == 0)
            def _():
                acc[i] = bufb[cur, i]

            @pl.when(ti > 0)
            def _():
                acc[i] = acc[i] + bufb[cur, i]
            return 0
        lax.fori_loop(0, BPW, addrow, 0)
        return 0
    lax.fori_loop(0, 10, title_body, 0)

    def scalerow(i, _):
        acc[i] = acc[i] * 0.1
        return 0
    lax.fori_loop(0, BPW, scalerow, 0)
    pltpu.sync_copy(acc, out.at[pl.ds(base, BPW), 63])


def _gather_fields(x, emb1, pair_tables, emb3, title_table):
    xi_t = x[:, :28].astype(jnp.int32).T
    tables = ([pt[0] for pt in pair_tables] + [pt[1] for pt in pair_tables]
              + list(emb1) + list(emb3) + [title_table])
    pairs = [(i, j) for i in range(9) for j in range(i, 9)]
    cols = ([i for i, _ in pairs] + [j for _, j in pairs]
            + list(range(9)) + list(range(9, 18)) + list(range(18, 28)))
    sizes = [int(t.shape[0]) for t in tables]
    offs = np.concatenate([[0], np.cumsum(sizes[:-1])]).astype(np.int32)
    row_tbl = list(range(108)) + [108] * 10
    row_off = offs[np.array(row_tbl)]
    big = jnp.concatenate(tables, axis=0)
    idx = xi_t[np.array(cols)] + jnp.asarray(row_off)[:, None]
    return _sc_fields(big, idx)


def _va_body(xd_ref, vw_ref, vb_ref, aw_ref, ab_ref, o_ref):
    xd = xd_ref[...]
    va = jnp.dot(xd[:, :128], vw_ref[...], preferred_element_type=jnp.float32)
    aa = jnp.dot(xd[:, 128:], aw_ref[...], preferred_element_type=jnp.float32)
    o_ref[...] = jnp.concatenate([va + vb_ref[...], aa + ab_ref[...]], axis=1)


def _final_body(x_ref, w_ref, b_ref, o_ref):
    o_ref[...] = jnp.sum(x_ref[...] * w_ref[...], axis=1, keepdims=True) + b_ref[0, 0]


def kernel(x, emb1, pair_tables, emb3, title_table, video_W, video_b,
           audio_W, audio_b, att1, att2, lin_W, lin_b):
    fields = _gather_fields(x, emb1, pair_tables, emb3, title_table)

    va = pl.pallas_call(
        _va_body,
        out_shape=jax.ShapeDtypeStruct((B, 2 * EMB), jnp.float32),
    )(x[:, 28:284], video_W.T, video_b[None, :], audio_W.T, audio_b[None, :])

    xf = jnp.concatenate([fields, va.reshape(B, 2, EMB)], axis=1).reshape(ROWS, EMB)

    xf = _attention_layer(xf, att1)
    xf = _attention_layer(xf, att2)
    xf = _attention_layer(xf, att2)

    return pl.pallas_call(
        _final_body,
        out_shape=jax.ShapeDtypeStruct((B, 1), jnp.float32),
    )(xf.reshape(B, M * EMB), lin_W, lin_b[None, :])
